# baseline (speedup 1.0000x reference)
import functools

import jax
import jax.numpy as jnp
from jax.experimental import pallas as pl
from jax.experimental.pallas import tpu as pltpu

TEMP = 0.2
EPS = 1e-12


def _body(x_hbm, pt_hbm, sa_hbm, s0_hbm, out_ref,
          pn_ref, ptv_ref, sav_ref, s0v_ref, xbuf_ref, m_ref, sem,
          *, n_k, ck, g):
    b, n, d = x_hbm.shape
    nchunks = b // g

    def xcopy(chunk, buf):
        return pltpu.make_async_copy(
            x_hbm.at[pl.ds(chunk * g, g)], xbuf_ref.at[buf], sem.at[buf])

    xcopy(0, 0).start()
    c1 = pltpu.make_async_copy(pt_hbm, ptv_ref, sem.at[2])
    c2 = pltpu.make_async_copy(sa_hbm, sav_ref, sem.at[3])
    c3 = pltpu.make_async_copy(s0_hbm, s0v_ref, sem.at[4])
    c1.start()
    c2.start()
    c3.start()
    c1.wait()
    p = ptv_ref[...]
    pn2 = jnp.sum(p * p, axis=1, keepdims=True)
    pinv = 1.0 / jnp.maximum(jnp.sqrt(pn2), EPS)
    pn_ref[...] = jnp.transpose((p * pinv).astype(jnp.float8_e4m3fn))

    for i in range(nchunks):
        slot = i % 2
        if i + 1 < nchunks:
            xcopy(i + 1, 1 - slot).start()
        xcopy(i, slot).wait()
        xb = xbuf_ref[slot].reshape(g * n, d)
        n2 = jnp.sum(xb * xb, axis=1, keepdims=True)
        inv = 1.0 / jnp.maximum(jnp.sqrt(n2), EPS)
        xn = (xb * inv).astype(jnp.float8_e4m3fn)
        logits = jnp.dot(xn, pn_ref[...], preferred_element_type=jnp.float32)
        m_ref[i * g:(i + 1) * g, :] = jnp.max(logits.reshape(g, n, ck), axis=1)

    c2.wait()
    c3.wait()
    sa = sav_ref[...]
    e = jnp.exp(sa - jnp.max(sa))
    s0 = s0v_ref[...]
    me = (m_ref[...] * e).astype(jnp.bfloat16)
    num = jnp.dot(me, s0, preferred_element_type=jnp.float32)
    den = jnp.dot(e.astype(jnp.bfloat16), s0,
                  preferred_element_type=jnp.float32)
    out_ref[...] = num * (float(n_k) / TEMP / jnp.maximum(den, 1e-30))


def kernel(x, prototypes, sa_weights):
    b, n, d = x.shape
    c, n_k, _ = prototypes.shape
    n_classes = c - 1
    ck = 1024
    cp = 256
    g = 16

    pt = jnp.pad(prototypes.reshape(c * n_k, d),
                 ((0, ck - c * n_k), (0, 0)))
    sa = jnp.pad(sa_weights.reshape(1, n_classes * n_k),
                 ((0, 0), (0, ck - n_classes * n_k)), constant_values=-1e30)
    s0 = (jax.lax.broadcasted_iota(jnp.int32, (ck, cp), 0) // n_k
          == jax.lax.broadcasted_iota(jnp.int32, (ck, cp), 1)
          ).astype(jnp.bfloat16)

    hbm = pl.BlockSpec(memory_space=pltpu.MemorySpace.HBM)
    out = pl.pallas_call(
        functools.partial(_body, n_k=n_k, ck=ck, g=g),
        in_specs=[hbm, hbm, hbm, hbm],
        out_specs=pl.BlockSpec((b, cp), lambda: (0, 0)),
        out_shape=jax.ShapeDtypeStruct((b, cp), jnp.float32),
        scratch_shapes=[pltpu.VMEM((d, ck), jnp.float8_e4m3fn),
                        pltpu.VMEM((ck, d), jnp.float32),
                        pltpu.VMEM((1, ck), jnp.float32),
                        pltpu.VMEM((ck, cp), jnp.bfloat16),
                        pltpu.VMEM((2, g, n, d), jnp.float32),
                        pltpu.VMEM((b, ck), jnp.float32),
                        pltpu.SemaphoreType.DMA((5,))],
    )(x, pt, sa, s0)
    return out[:, :n_classes]


# speedup vs baseline: 1.1037x; 1.1037x over previous
import functools

import jax
import jax.numpy as jnp
from jax.experimental import pallas as pl
from jax.experimental.pallas import tpu as pltpu

TEMP = 0.2
EPS = 1e-12


def _body(x_hbm, pt_hbm, sa_hbm, s0_hbm, out_ref,
          pn_ref, ptv_ref, sav_ref, s0v_ref, xbuf_ref, m_ref, sem,
          *, n_k, ck, g):
    b, n, d = x_hbm.shape
    nchunks = b // g

    def xcopy(chunk, buf):
        return pltpu.make_async_copy(
            x_hbm.at[pl.ds(chunk * g, g)], xbuf_ref.at[buf], sem.at[buf])

    xcopy(0, 0).start()
    c1 = pltpu.make_async_copy(pt_hbm, ptv_ref, sem.at[2])
    c2 = pltpu.make_async_copy(sa_hbm, sav_ref, sem.at[3])
    c3 = pltpu.make_async_copy(s0_hbm, s0v_ref, sem.at[4])
    c1.start()
    c2.start()
    c3.start()
    c1.wait()
    p = ptv_ref[...]
    pn2 = jnp.sum(p * p, axis=1, keepdims=True)
    pinv = 1.0 / jnp.maximum(jnp.sqrt(pn2), EPS)
    pn_ref[...] = jnp.transpose((p * pinv).astype(jnp.float8_e4m3fn))

    for i in range(nchunks):
        slot = i % 2
        if i + 1 < nchunks:
            xcopy(i + 1, 1 - slot).start()
        xcopy(i, slot).wait()
        xb = xbuf_ref[slot].reshape(g * n, d)
        n2 = jnp.sum(xb * xb, axis=1, keepdims=True)
        inv = 1.0 / jnp.maximum(jnp.sqrt(n2), EPS)
        xn = (xb * inv).astype(jnp.float8_e4m3fn)
        logits = jnp.dot(xn, pn_ref[...], preferred_element_type=jnp.float32)
        m_ref[i * g:(i + 1) * g, :] = jnp.max(logits.reshape(g, n, ck), axis=1)

    c2.wait()
    c3.wait()
    sa = sav_ref[...]
    e = jnp.exp(sa - jnp.max(sa))
    s0 = s0v_ref[...]
    me = (m_ref[...] * e).astype(jnp.bfloat16)
    num = jnp.dot(me, s0, preferred_element_type=jnp.float32)
    den = jnp.dot(e.astype(jnp.bfloat16), s0,
                  preferred_element_type=jnp.float32)
    out_ref[...] = num * (float(n_k) / TEMP / jnp.maximum(den, 1e-30))


def kernel(x, prototypes, sa_weights):
    b, n, d = x.shape
    c, n_k, _ = prototypes.shape
    n_classes = c - 1
    ck = 1024
    cp = 256
    g = 4

    pt = jnp.pad(prototypes.reshape(c * n_k, d),
                 ((0, ck - c * n_k), (0, 0)))
    sa = jnp.pad(sa_weights.reshape(1, n_classes * n_k),
                 ((0, 0), (0, ck - n_classes * n_k)), constant_values=-1e30)
    s0 = (jax.lax.broadcasted_iota(jnp.int32, (ck, cp), 0) // n_k
          == jax.lax.broadcasted_iota(jnp.int32, (ck, cp), 1)
          ).astype(jnp.bfloat16)

    hbm = pl.BlockSpec(memory_space=pltpu.MemorySpace.HBM)
    out = pl.pallas_call(
        functools.partial(_body, n_k=n_k, ck=ck, g=g),
        in_specs=[hbm, hbm, hbm, hbm],
        out_specs=pl.BlockSpec((b, cp), lambda: (0, 0)),
        out_shape=jax.ShapeDtypeStruct((b, cp), jnp.float32),
        scratch_shapes=[pltpu.VMEM((d, ck), jnp.float8_e4m3fn),
                        pltpu.VMEM((ck, d), jnp.float32),
                        pltpu.VMEM((1, ck), jnp.float32),
                        pltpu.VMEM((ck, cp), jnp.bfloat16),
                        pltpu.VMEM((2, g, n, d), jnp.float32),
                        pltpu.VMEM((b, ck), jnp.float32),
                        pltpu.SemaphoreType.DMA((5,))],
    )(x, pt, sa, s0)
    return out[:, :n_classes]
